# Initial kernel scaffold; baseline (speedup 1.0000x reference)
import jax
import jax.numpy as jnp
from jax.experimental import pallas as pl


def kernel(feat_map, W_quat, b_quat, W_trans, b_trans, W_embed, b_embed, W_conf, b_conf, W_bbox, b_bbox):
    raise NotImplementedError("write your pallas kernel here")


# speedup vs baseline: 1.4529x; 1.4529x over previous
import functools

import jax
import jax.numpy as jnp
from jax.experimental import pallas as pl
from jax.experimental.pallas import tpu as pltpu

_Q0, _T0, _E0, _C0, _B0 = 0, 8, 16, 48, 56
_NB = 2048


def _head_kernel(f_ref, w_ref, b_ref, quat_ref, trans_ref, embed_ref,
                 conf_ref, bbox_ref, r_ref):
    f = f_ref[0]
    out = jnp.dot(w_ref[...], f, preferred_element_type=jnp.float32)
    out = out + b_ref[...]

    q8 = out[_Q0:_Q0 + 8]
    s = jnp.sum(q8 * q8, axis=0, keepdims=True)
    inv = 1.0 / (jnp.sqrt(s) + 1e-8)
    qn = q8 * inv
    quat_ref[0] = qn[0:4]

    trans_ref[0] = out[_T0:_T0 + 3]
    embed_ref[0] = out[_E0:_E0 + 32]
    conf_ref[0] = jax.nn.sigmoid(out[_C0:_C0 + 1])
    bbox_ref[0] = out[_B0:_B0 + 4]

    w = qn[0:1]
    x = qn[1:2]
    y = qn[2:3]
    z = qn[3:4]
    xx = 2.0 * x * x
    yy = 2.0 * y * y
    zz = 2.0 * z * z
    xy = 2.0 * x * y
    xz = 2.0 * x * z
    yz = 2.0 * y * z
    xw = 2.0 * x * w
    yw = 2.0 * y * w
    zw = 2.0 * z * w
    r00 = 1.0 - (yy + zz)
    r01 = xy - zw
    r02 = xz + yw
    r10 = xy + zw
    r11 = 1.0 - (xx + zz)
    r12 = yz - xw
    r20 = xz - yw
    r21 = yz + xw
    r22 = 1.0 - (xx + yy)
    r_ref[0] = jnp.concatenate(
        [r00, r01, r02, r10, r11, r12, r20, r21, r22], axis=0)


@functools.partial(jax.jit, static_argnames=())
def kernel(feat_map, W_quat, b_quat, W_trans, b_trans, W_embed, b_embed,
           W_conf, b_conf, W_bbox, b_bbox):
    B, C, H, W = feat_map.shape
    HW = H * W
    feat2d = feat_map.reshape(B, C, HW)

    wall = jnp.zeros((64, C), jnp.float32)
    wall = wall.at[_Q0:_Q0 + 4].set(W_quat)
    wall = wall.at[_T0:_T0 + 3].set(W_trans)
    wall = wall.at[_E0:_E0 + 32].set(W_embed)
    wall = wall.at[_C0:_C0 + 1].set(W_conf)
    wall = wall.at[_B0:_B0 + 4].set(W_bbox)
    ball = jnp.zeros((64, 1), jnp.float32)
    ball = ball.at[_Q0:_Q0 + 4, 0].set(b_quat)
    ball = ball.at[_T0:_T0 + 3, 0].set(b_trans)
    ball = ball.at[_E0:_E0 + 32, 0].set(b_embed)
    ball = ball.at[_C0:_C0 + 1, 0].set(b_conf)
    ball = ball.at[_B0:_B0 + 4, 0].set(b_bbox)

    grid = (B, HW // _NB)
    px_spec = lambda o: pl.BlockSpec((1, o, _NB), lambda b, i: (b, 0, i))
    quat_o, trans_o, embed_o, conf_o, bbox_o, r_o = pl.pallas_call(
        _head_kernel,
        grid=grid,
        in_specs=[
            pl.BlockSpec((1, C, _NB), lambda b, i: (b, 0, i)),
            pl.BlockSpec((64, C), lambda b, i: (0, 0)),
            pl.BlockSpec((64, 1), lambda b, i: (0, 0)),
        ],
        out_specs=[
            px_spec(4), px_spec(3), px_spec(32), px_spec(1), px_spec(4),
            px_spec(9),
        ],
        out_shape=[
            jax.ShapeDtypeStruct((B, 4, HW), jnp.float32),
            jax.ShapeDtypeStruct((B, 3, HW), jnp.float32),
            jax.ShapeDtypeStruct((B, 32, HW), jnp.float32),
            jax.ShapeDtypeStruct((B, 1, HW), jnp.float32),
            jax.ShapeDtypeStruct((B, 4, HW), jnp.float32),
            jax.ShapeDtypeStruct((B, 9, HW), jnp.float32),
        ],
        compiler_params=pltpu.CompilerParams(
            dimension_semantics=("parallel", "parallel")),
    )(feat2d, wall, ball)

    quat = quat_o.reshape(B, 4, H, W)
    trans = trans_o.reshape(B, 3, H, W)
    embed = embed_o.reshape(B, 32, H, W)
    conf = conf_o.reshape(B, 1, H, W)
    bbox = bbox_o.reshape(B, 4, H, W)
    global_R = r_o.transpose(0, 2, 1).reshape(B, H, W, 3, 3)
    return (quat, trans, embed, conf, bbox, global_R, trans)


# speedup vs baseline: 2.6580x; 1.8294x over previous
import functools

import jax
import jax.numpy as jnp
from jax.experimental import pallas as pl
from jax.experimental.pallas import tpu as pltpu

_Q0, _T0, _E0, _C0, _B0 = 0, 8, 16, 48, 56
_HB = 16
_NB = _HB * 128


def _head_kernel(f_ref, w_ref, b_ref, quat_ref, trans_ref, embed_ref,
                 conf_ref, bbox_ref, *r_refs):
    f = f_ref[0].reshape(64, _NB)
    out = jnp.dot(w_ref[...], f, preferred_element_type=jnp.float32)
    out = out + b_ref[...]

    q8 = out[_Q0:_Q0 + 8]
    s = jnp.sum(q8 * q8, axis=0, keepdims=True)
    inv = 1.0 / (jnp.sqrt(s) + 1e-8)
    qn = q8 * inv
    quat_ref[0] = qn[0:4].reshape(4, _HB, 128)

    trans_ref[0] = out[_T0:_T0 + 3].reshape(3, _HB, 128)
    embed_ref[0] = out[_E0:_E0 + 32].reshape(32, _HB, 128)
    conf_ref[0] = jax.nn.sigmoid(out[_C0:_C0 + 1]).reshape(1, _HB, 128)
    bbox_ref[0] = out[_B0:_B0 + 4].reshape(4, _HB, 128)

    w = qn[0:1]
    x = qn[1:2]
    y = qn[2:3]
    z = qn[3:4]
    xx = 2.0 * x * x
    yy = 2.0 * y * y
    zz = 2.0 * z * z
    xy = 2.0 * x * y
    xz = 2.0 * x * z
    yz = 2.0 * y * z
    xw = 2.0 * x * w
    yw = 2.0 * y * w
    zw = 2.0 * z * w
    rs = (1.0 - (yy + zz), xy - zw, xz + yw,
          xy + zw, 1.0 - (xx + zz), yz - xw,
          xz - yw, yz + xw, 1.0 - (xx + yy))
    for r_ref, r in zip(r_refs, rs):
        r_ref[0] = r.reshape(_HB, 128)


@functools.partial(jax.jit, static_argnames=())
def kernel(feat_map, W_quat, b_quat, W_trans, b_trans, W_embed, b_embed,
           W_conf, b_conf, W_bbox, b_bbox):
    B, C, H, W = feat_map.shape

    wall = jnp.zeros((64, C), jnp.float32)
    wall = wall.at[_Q0:_Q0 + 4].set(W_quat)
    wall = wall.at[_T0:_T0 + 3].set(W_trans)
    wall = wall.at[_E0:_E0 + 32].set(W_embed)
    wall = wall.at[_C0:_C0 + 1].set(W_conf)
    wall = wall.at[_B0:_B0 + 4].set(W_bbox)
    ball = jnp.zeros((64, 1), jnp.float32)
    ball = ball.at[_Q0:_Q0 + 4, 0].set(b_quat)
    ball = ball.at[_T0:_T0 + 3, 0].set(b_trans)
    ball = ball.at[_E0:_E0 + 32, 0].set(b_embed)
    ball = ball.at[_C0:_C0 + 1, 0].set(b_conf)
    ball = ball.at[_B0:_B0 + 4, 0].set(b_bbox)

    grid = (B, H // _HB)
    ch_spec = lambda o: pl.BlockSpec((1, o, _HB, 128), lambda b, i: (b, 0, i, 0))
    pl_spec = pl.BlockSpec((1, _HB, 128), lambda b, i: (b, i, 0))
    outs = pl.pallas_call(
        _head_kernel,
        grid=grid,
        in_specs=[
            ch_spec(C),
            pl.BlockSpec((64, C), lambda b, i: (0, 0)),
            pl.BlockSpec((64, 1), lambda b, i: (0, 0)),
        ],
        out_specs=[ch_spec(4), ch_spec(3), ch_spec(32), ch_spec(1), ch_spec(4)]
                  + [pl_spec] * 9,
        out_shape=[
            jax.ShapeDtypeStruct((B, 4, H, W), jnp.float32),
            jax.ShapeDtypeStruct((B, 3, H, W), jnp.float32),
            jax.ShapeDtypeStruct((B, 32, H, W), jnp.float32),
            jax.ShapeDtypeStruct((B, 1, H, W), jnp.float32),
            jax.ShapeDtypeStruct((B, 4, H, W), jnp.float32),
        ] + [jax.ShapeDtypeStruct((B, H, W), jnp.float32)] * 9,
        compiler_params=pltpu.CompilerParams(
            dimension_semantics=("parallel", "parallel")),
    )(feat_map, wall, ball)

    quat, trans, embed, conf, bbox = outs[:5]
    r00, r01, r02, r10, r11, r12, r20, r21, r22 = outs[5:]
    row0 = jnp.stack([r00, r01, r02], axis=-1)
    row1 = jnp.stack([r10, r11, r12], axis=-1)
    row2 = jnp.stack([r20, r21, r22], axis=-1)
    global_R = jnp.stack([row0, row1, row2], axis=-2)
    return (quat, trans, embed, conf, bbox, global_R, trans)


# speedup vs baseline: 3.9091x; 1.4707x over previous
import functools

import jax
import jax.numpy as jnp
from jax.experimental import pallas as pl
from jax.experimental.pallas import tpu as pltpu

_Q0, _T0, _E0, _C0, _B0 = 0, 8, 16, 48, 56
_HB = 64
_NB = _HB * 128


def _head_kernel(f_ref, w_ref, b_ref, quat_ref, trans_ref, embed_ref,
                 conf_ref, bbox_ref, *r_refs):
    f = f_ref[0].reshape(64, _NB)
    out = jnp.dot(w_ref[...], f, preferred_element_type=jnp.float32)
    out = out + b_ref[...]

    q8 = out[_Q0:_Q0 + 8]
    s = jnp.sum(q8 * q8, axis=0, keepdims=True)
    inv = 1.0 / (jnp.sqrt(s) + 1e-8)
    qn = q8 * inv
    quat_ref[0] = qn[0:4].reshape(4, _HB, 128)

    trans_ref[0] = out[_T0:_T0 + 3].reshape(3, _HB, 128)
    embed_ref[0] = out[_E0:_E0 + 32].reshape(32, _HB, 128)
    conf_ref[0] = jax.nn.sigmoid(out[_C0:_C0 + 1]).reshape(1, _HB, 128)
    bbox_ref[0] = out[_B0:_B0 + 4].reshape(4, _HB, 128)

    w = qn[0:1]
    x = qn[1:2]
    y = qn[2:3]
    z = qn[3:4]
    xx = 2.0 * x * x
    yy = 2.0 * y * y
    zz = 2.0 * z * z
    xy = 2.0 * x * y
    xz = 2.0 * x * z
    yz = 2.0 * y * z
    xw = 2.0 * x * w
    yw = 2.0 * y * w
    zw = 2.0 * z * w
    rs = (1.0 - (yy + zz), xy - zw, xz + yw,
          xy + zw, 1.0 - (xx + zz), yz - xw,
          xz - yw, yz + xw, 1.0 - (xx + yy))
    for r_ref, r in zip(r_refs, rs):
        r_ref[0] = r.reshape(_HB, 128)


@functools.partial(jax.jit, static_argnames=())
def kernel(feat_map, W_quat, b_quat, W_trans, b_trans, W_embed, b_embed,
           W_conf, b_conf, W_bbox, b_bbox):
    B, C, H, W = feat_map.shape

    wall = jnp.zeros((64, C), jnp.float32)
    wall = wall.at[_Q0:_Q0 + 4].set(W_quat)
    wall = wall.at[_T0:_T0 + 3].set(W_trans)
    wall = wall.at[_E0:_E0 + 32].set(W_embed)
    wall = wall.at[_C0:_C0 + 1].set(W_conf)
    wall = wall.at[_B0:_B0 + 4].set(W_bbox)
    ball = jnp.zeros((64, 1), jnp.float32)
    ball = ball.at[_Q0:_Q0 + 4, 0].set(b_quat)
    ball = ball.at[_T0:_T0 + 3, 0].set(b_trans)
    ball = ball.at[_E0:_E0 + 32, 0].set(b_embed)
    ball = ball.at[_C0:_C0 + 1, 0].set(b_conf)
    ball = ball.at[_B0:_B0 + 4, 0].set(b_bbox)

    grid = (B, H // _HB)
    ch_spec = lambda o: pl.BlockSpec((1, o, _HB, 128), lambda b, i: (b, 0, i, 0))
    pl_spec = pl.BlockSpec((1, _HB, 128), lambda b, i: (b, i, 0))
    outs = pl.pallas_call(
        _head_kernel,
        grid=grid,
        in_specs=[
            ch_spec(C),
            pl.BlockSpec((64, C), lambda b, i: (0, 0)),
            pl.BlockSpec((64, 1), lambda b, i: (0, 0)),
        ],
        out_specs=[ch_spec(4), ch_spec(3), ch_spec(32), ch_spec(1), ch_spec(4)]
                  + [pl_spec] * 9,
        out_shape=[
            jax.ShapeDtypeStruct((B, 4, H, W), jnp.float32),
            jax.ShapeDtypeStruct((B, 3, H, W), jnp.float32),
            jax.ShapeDtypeStruct((B, 32, H, W), jnp.float32),
            jax.ShapeDtypeStruct((B, 1, H, W), jnp.float32),
            jax.ShapeDtypeStruct((B, 4, H, W), jnp.float32),
        ] + [jax.ShapeDtypeStruct((B, H, W), jnp.float32)] * 9,
        compiler_params=pltpu.CompilerParams(
            dimension_semantics=("parallel", "parallel")),
    )(feat_map, wall, ball)

    quat, trans, embed, conf, bbox = outs[:5]
    r00, r01, r02, r10, r11, r12, r20, r21, r22 = outs[5:]
    row0 = jnp.stack([r00, r01, r02], axis=-1)
    row1 = jnp.stack([r10, r11, r12], axis=-1)
    row2 = jnp.stack([r20, r21, r22], axis=-1)
    global_R = jnp.stack([row0, row1, row2], axis=-2)
    return (quat, trans, embed, conf, bbox, global_R, trans)


# speedup vs baseline: 4.2230x; 1.0803x over previous
import functools

import jax
import jax.numpy as jnp
from jax.experimental import pallas as pl
from jax.experimental.pallas import tpu as pltpu

_Q0, _T0, _E0, _C0, _B0 = 0, 8, 16, 48, 56
_HB = 128
_NB = _HB * 128


def _head_kernel(f_ref, w_ref, b_ref, quat_ref, trans_ref, embed_ref,
                 conf_ref, bbox_ref, *r_refs):
    f = f_ref[0].reshape(64, _NB)
    out = jnp.dot(w_ref[...], f, preferred_element_type=jnp.float32)
    out = out + b_ref[...]

    q8 = out[_Q0:_Q0 + 8]
    s = jnp.sum(q8 * q8, axis=0, keepdims=True)
    inv = 1.0 / (jnp.sqrt(s) + 1e-8)
    qn = q8 * inv
    quat_ref[0] = qn[0:4].reshape(4, _HB, 128)

    trans_ref[0] = out[_T0:_T0 + 3].reshape(3, _HB, 128)
    embed_ref[0] = out[_E0:_E0 + 32].reshape(32, _HB, 128)
    conf_ref[0] = jax.nn.sigmoid(out[_C0:_C0 + 1]).reshape(1, _HB, 128)
    bbox_ref[0] = out[_B0:_B0 + 4].reshape(4, _HB, 128)

    w = qn[0:1]
    x = qn[1:2]
    y = qn[2:3]
    z = qn[3:4]
    xx = 2.0 * x * x
    yy = 2.0 * y * y
    zz = 2.0 * z * z
    xy = 2.0 * x * y
    xz = 2.0 * x * z
    yz = 2.0 * y * z
    xw = 2.0 * x * w
    yw = 2.0 * y * w
    zw = 2.0 * z * w
    rs = (1.0 - (yy + zz), xy - zw, xz + yw,
          xy + zw, 1.0 - (xx + zz), yz - xw,
          xz - yw, yz + xw, 1.0 - (xx + yy))
    for r_ref, r in zip(r_refs, rs):
        r_ref[0] = r.reshape(_HB, 128)


@functools.partial(jax.jit, static_argnames=())
def kernel(feat_map, W_quat, b_quat, W_trans, b_trans, W_embed, b_embed,
           W_conf, b_conf, W_bbox, b_bbox):
    B, C, H, W = feat_map.shape

    wall = jnp.zeros((64, C), jnp.float32)
    wall = wall.at[_Q0:_Q0 + 4].set(W_quat)
    wall = wall.at[_T0:_T0 + 3].set(W_trans)
    wall = wall.at[_E0:_E0 + 32].set(W_embed)
    wall = wall.at[_C0:_C0 + 1].set(W_conf)
    wall = wall.at[_B0:_B0 + 4].set(W_bbox)
    ball = jnp.zeros((64, 1), jnp.float32)
    ball = ball.at[_Q0:_Q0 + 4, 0].set(b_quat)
    ball = ball.at[_T0:_T0 + 3, 0].set(b_trans)
    ball = ball.at[_E0:_E0 + 32, 0].set(b_embed)
    ball = ball.at[_C0:_C0 + 1, 0].set(b_conf)
    ball = ball.at[_B0:_B0 + 4, 0].set(b_bbox)

    grid = (B, H // _HB)
    ch_spec = lambda o: pl.BlockSpec((1, o, _HB, 128), lambda b, i: (b, 0, i, 0))
    pl_spec = pl.BlockSpec((1, _HB, 128), lambda b, i: (b, i, 0))
    outs = pl.pallas_call(
        _head_kernel,
        grid=grid,
        in_specs=[
            ch_spec(C),
            pl.BlockSpec((64, C), lambda b, i: (0, 0)),
            pl.BlockSpec((64, 1), lambda b, i: (0, 0)),
        ],
        out_specs=[ch_spec(4), ch_spec(3), ch_spec(32), ch_spec(1), ch_spec(4)]
                  + [pl_spec] * 9,
        out_shape=[
            jax.ShapeDtypeStruct((B, 4, H, W), jnp.float32),
            jax.ShapeDtypeStruct((B, 3, H, W), jnp.float32),
            jax.ShapeDtypeStruct((B, 32, H, W), jnp.float32),
            jax.ShapeDtypeStruct((B, 1, H, W), jnp.float32),
            jax.ShapeDtypeStruct((B, 4, H, W), jnp.float32),
        ] + [jax.ShapeDtypeStruct((B, H, W), jnp.float32)] * 9,
        compiler_params=pltpu.CompilerParams(
            dimension_semantics=("parallel", "parallel")),
    )(feat_map, wall, ball)

    quat, trans, embed, conf, bbox = outs[:5]
    r00, r01, r02, r10, r11, r12, r20, r21, r22 = outs[5:]
    row0 = jnp.stack([r00, r01, r02], axis=-1)
    row1 = jnp.stack([r10, r11, r12], axis=-1)
    row2 = jnp.stack([r20, r21, r22], axis=-1)
    global_R = jnp.stack([row0, row1, row2], axis=-2)
    return (quat, trans, embed, conf, bbox, global_R, trans)


# speedup vs baseline: 4.2250x; 1.0005x over previous
import functools

import jax
import jax.numpy as jnp
from jax.experimental import pallas as pl
from jax.experimental.pallas import tpu as pltpu

_Q0, _T0, _E0, _C0, _B0 = 0, 8, 16, 48, 56
_HB = 128
_NB = _HB * 128


def _head_kernel(f_ref, f2_ref, w_ref, b_ref, quat_ref, trans_ref, embed_ref,
                 conf_ref, bbox_ref, *r_refs):
    f = jnp.concatenate([f_ref[0], f2_ref[0]], axis=0).reshape(64, _NB)
    out = jnp.dot(w_ref[...], f, preferred_element_type=jnp.float32)
    out = out + b_ref[...]

    q8 = out[_Q0:_Q0 + 8]
    s = jnp.sum(q8 * q8, axis=0, keepdims=True)
    inv = 1.0 / (jnp.sqrt(s) + 1e-8)
    qn = q8 * inv
    quat_ref[0] = qn[0:4].reshape(4, _HB, 128)

    trans_ref[0] = out[_T0:_T0 + 3].reshape(3, _HB, 128)
    embed_ref[0] = out[_E0:_E0 + 32].reshape(32, _HB, 128)
    conf_ref[0] = jax.nn.sigmoid(out[_C0:_C0 + 1]).reshape(1, _HB, 128)
    bbox_ref[0] = out[_B0:_B0 + 4].reshape(4, _HB, 128)

    w = qn[0:1]
    x = qn[1:2]
    y = qn[2:3]
    z = qn[3:4]
    xx = 2.0 * x * x
    yy = 2.0 * y * y
    zz = 2.0 * z * z
    xy = 2.0 * x * y
    xz = 2.0 * x * z
    yz = 2.0 * y * z
    xw = 2.0 * x * w
    yw = 2.0 * y * w
    zw = 2.0 * z * w
    rs = (1.0 - (yy + zz), xy - zw, xz + yw,
          xy + zw, 1.0 - (xx + zz), yz - xw,
          xz - yw, yz + xw, 1.0 - (xx + yy))
    for r_ref, r in zip(r_refs, rs):
        r_ref[0] = r.reshape(_HB, 128)


@functools.partial(jax.jit, static_argnames=())
def kernel(feat_map, W_quat, b_quat, W_trans, b_trans, W_embed, b_embed,
           W_conf, b_conf, W_bbox, b_bbox):
    B, C, H, W = feat_map.shape

    wall = jnp.zeros((64, C), jnp.float32)
    wall = wall.at[_Q0:_Q0 + 4].set(W_quat)
    wall = wall.at[_T0:_T0 + 3].set(W_trans)
    wall = wall.at[_E0:_E0 + 32].set(W_embed)
    wall = wall.at[_C0:_C0 + 1].set(W_conf)
    wall = wall.at[_B0:_B0 + 4].set(W_bbox)
    ball = jnp.zeros((64, 1), jnp.float32)
    ball = ball.at[_Q0:_Q0 + 4, 0].set(b_quat)
    ball = ball.at[_T0:_T0 + 3, 0].set(b_trans)
    ball = ball.at[_E0:_E0 + 32, 0].set(b_embed)
    ball = ball.at[_C0:_C0 + 1, 0].set(b_conf)
    ball = ball.at[_B0:_B0 + 4, 0].set(b_bbox)

    grid = (B, H // _HB)
    ch_spec = lambda o: pl.BlockSpec((1, o, _HB, 128), lambda b, i: (b, 0, i, 0))
    pl_spec = pl.BlockSpec((1, _HB, 128), lambda b, i: (b, i, 0))
    outs = pl.pallas_call(
        _head_kernel,
        grid=grid,
        in_specs=[
            pl.BlockSpec((1, C // 2, _HB, 128), lambda b, i: (b, 0, i, 0)),
            pl.BlockSpec((1, C // 2, _HB, 128), lambda b, i: (b, 1, i, 0)),
            pl.BlockSpec((64, C), lambda b, i: (0, 0)),
            pl.BlockSpec((64, 1), lambda b, i: (0, 0)),
        ],
        out_specs=[ch_spec(4), ch_spec(3), ch_spec(32), ch_spec(1), ch_spec(4)]
                  + [pl_spec] * 9,
        out_shape=[
            jax.ShapeDtypeStruct((B, 4, H, W), jnp.float32),
            jax.ShapeDtypeStruct((B, 3, H, W), jnp.float32),
            jax.ShapeDtypeStruct((B, 32, H, W), jnp.float32),
            jax.ShapeDtypeStruct((B, 1, H, W), jnp.float32),
            jax.ShapeDtypeStruct((B, 4, H, W), jnp.float32),
        ] + [jax.ShapeDtypeStruct((B, H, W), jnp.float32)] * 9,
        compiler_params=pltpu.CompilerParams(
            dimension_semantics=("parallel", "parallel")),
    )(feat_map, feat_map, wall, ball)

    quat, trans, embed, conf, bbox = outs[:5]
    r00, r01, r02, r10, r11, r12, r20, r21, r22 = outs[5:]
    row0 = jnp.stack([r00, r01, r02], axis=-1)
    row1 = jnp.stack([r10, r11, r12], axis=-1)
    row2 = jnp.stack([r20, r21, r22], axis=-1)
    global_R = jnp.stack([row0, row1, row2], axis=-2)
    return (quat, trans, embed, conf, bbox, global_R, trans)


# speedup vs baseline: 4.7287x; 1.1192x over previous
import functools

import jax
import jax.numpy as jnp
from jax.experimental import pallas as pl
from jax.experimental.pallas import tpu as pltpu

_Q0, _T0, _E0, _C0, _B0 = 0, 8, 16, 48, 56
_HB = 128
_NB = _HB * 128


def _head_kernel(f_ref, w_ref, b_ref, quat_ref, trans_ref, embed_ref,
                 conf_ref, bbox_ref, *r_refs):
    f = f_ref[0].reshape(64, _NB)
    out = jnp.dot(w_ref[...], f, preferred_element_type=jnp.float32)
    out = out + b_ref[...]

    q4 = out[_Q0:_Q0 + 4].reshape(4, _HB, 128)
    trans_ref[0] = out[_T0:_T0 + 3].reshape(3, _HB, 128)
    embed_ref[0] = out[_E0:_E0 + 32].reshape(32, _HB, 128)
    conf_ref[0] = jax.nn.sigmoid(out[_C0:_C0 + 1].reshape(1, _HB, 128))
    bbox_ref[0] = out[_B0:_B0 + 4].reshape(4, _HB, 128)

    s = jnp.sum(q4 * q4, axis=0, keepdims=True)
    inv = 1.0 / (jnp.sqrt(s) + 1e-8)
    qn = q4 * inv
    quat_ref[0] = qn

    w = qn[0]
    x = qn[1]
    y = qn[2]
    z = qn[3]
    xx = 2.0 * x * x
    yy = 2.0 * y * y
    zz = 2.0 * z * z
    xy = 2.0 * x * y
    xz = 2.0 * x * z
    yz = 2.0 * y * z
    xw = 2.0 * x * w
    yw = 2.0 * y * w
    zw = 2.0 * z * w
    rs = (1.0 - (yy + zz), xy - zw, xz + yw,
          xy + zw, 1.0 - (xx + zz), yz - xw,
          xz - yw, yz + xw, 1.0 - (xx + yy))
    for r_ref, r in zip(r_refs, rs):
        r_ref[0] = r


@functools.partial(jax.jit, static_argnames=())
def kernel(feat_map, W_quat, b_quat, W_trans, b_trans, W_embed, b_embed,
           W_conf, b_conf, W_bbox, b_bbox):
    B, C, H, W = feat_map.shape

    wall = jnp.zeros((64, C), jnp.float32)
    wall = wall.at[_Q0:_Q0 + 4].set(W_quat)
    wall = wall.at[_T0:_T0 + 3].set(W_trans)
    wall = wall.at[_E0:_E0 + 32].set(W_embed)
    wall = wall.at[_C0:_C0 + 1].set(W_conf)
    wall = wall.at[_B0:_B0 + 4].set(W_bbox)
    ball = jnp.zeros((64, 1), jnp.float32)
    ball = ball.at[_Q0:_Q0 + 4, 0].set(b_quat)
    ball = ball.at[_T0:_T0 + 3, 0].set(b_trans)
    ball = ball.at[_E0:_E0 + 32, 0].set(b_embed)
    ball = ball.at[_C0:_C0 + 1, 0].set(b_conf)
    ball = ball.at[_B0:_B0 + 4, 0].set(b_bbox)

    grid = (B, H // _HB)
    ch_spec = lambda o: pl.BlockSpec((1, o, _HB, 128), lambda b, i: (b, 0, i, 0))
    pl_spec = pl.BlockSpec((1, _HB, 128), lambda b, i: (b, i, 0))
    outs = pl.pallas_call(
        _head_kernel,
        grid=grid,
        in_specs=[
            ch_spec(C),
            pl.BlockSpec((64, C), lambda b, i: (0, 0)),
            pl.BlockSpec((64, 1), lambda b, i: (0, 0)),
        ],
        out_specs=[ch_spec(4), ch_spec(3), ch_spec(32), ch_spec(1), ch_spec(4)]
                  + [pl_spec] * 9,
        out_shape=[
            jax.ShapeDtypeStruct((B, 4, H, W), jnp.float32),
            jax.ShapeDtypeStruct((B, 3, H, W), jnp.float32),
            jax.ShapeDtypeStruct((B, 32, H, W), jnp.float32),
            jax.ShapeDtypeStruct((B, 1, H, W), jnp.float32),
            jax.ShapeDtypeStruct((B, 4, H, W), jnp.float32),
        ] + [jax.ShapeDtypeStruct((B, H, W), jnp.float32)] * 9,
        compiler_params=pltpu.CompilerParams(
            dimension_semantics=("parallel", "parallel")),
    )(feat_map, wall, ball)

    quat, trans, embed, conf, bbox = outs[:5]
    r00, r01, r02, r10, r11, r12, r20, r21, r22 = outs[5:]
    row0 = jnp.stack([r00, r01, r02], axis=-1)
    row1 = jnp.stack([r10, r11, r12], axis=-1)
    row2 = jnp.stack([r20, r21, r22], axis=-1)
    global_R = jnp.stack([row0, row1, row2], axis=-2)
    return (quat, trans, embed, conf, bbox, global_R, trans)
